# Initial kernel scaffold; baseline (speedup 1.0000x reference)
import jax
import jax.numpy as jnp
from jax.experimental import pallas as pl


def kernel(h, edge_index, e, W_feat, b_feat, gamma_feat, beta_feat, W_pool, b_pool, gamma_pool, beta_pool):
    raise NotImplementedError("write your pallas kernel here")


# speedup vs baseline: 8.3904x; 8.3904x over previous
import functools

import jax
import jax.numpy as jnp
from jax import lax
from jax.experimental import pallas as pl
from jax.experimental.pallas import tpu as pltpu
from jax.experimental.pallas import tpu_sc as plsc

N = 10000
E = 320000
D = 128
K = 32
BZ = 100

NC = 2
NS = 16
NW = NC * NS
NPAD = 10240
CHUNK = 128
NCHUNKS = 2560
EPAD = NCHUNKS * CHUNK
CPW = NCHUNKS // NW
PAD_ROWS = NPAD - N


def _sc_mesh():
    return plsc.VectorSubcoreMesh(core_axis_name="c", subcore_axis_name="s")


def _deg_body(src_hbm, dst_hbm, out_hbm, sidx, didx, ones_v, zbuf,
              hist_out, hist_in):
    c = lax.axis_index("c")
    s = lax.axis_index("s")
    wid = s * NC + c

    def _zfill(i, _):
        i16 = pl.multiple_of(i * 16, 16)
        zbuf[pl.ds(i16, 16)] = jnp.zeros((16,), jnp.float32)
        return 0
    lax.fori_loop(0, 640 // 16, _zfill, 0)

    def _ofill(r, _):
        for i in range(CHUNK // 16):
            ones_v[r, pl.ds(i * 16, 16)] = jnp.ones((16,), jnp.float32)
        return 0
    lax.fori_loop(0, CPW, _ofill, 0)

    pltpu.sync_copy(zbuf, hist_out.at[pl.ds(s * 640, 640)])
    pltpu.sync_copy(zbuf, hist_in.at[pl.ds(s * 640, 640)])

    pltpu.sync_copy(src_hbm.at[pl.ds(wid * CPW, CPW)], sidx)
    pltpu.sync_copy(dst_hbm.at[pl.ds(wid * CPW, CPW)], didx)

    plsc.subcore_barrier()

    def _scat(j, _):
        pltpu.sync_copy(ones_v.at[j], hist_out.at[sidx.at[j]], add=True)
        pltpu.sync_copy(ones_v.at[j], hist_in.at[didx.at[j]], add=True)
        return 0
    lax.fori_loop(0, CPW, _scat, 0)

    plsc.subcore_barrier()

    pltpu.sync_copy(hist_out.at[pl.ds(s * 640, 640)],
                    out_hbm.at[c, 0, pl.ds(s * 640, 640)])
    pltpu.sync_copy(hist_in.at[pl.ds(s * 640, 640)],
                    out_hbm.at[c, 1, pl.ds(s * 640, 640)])


def _degrees(src2d, dst2d):
    f = pl.kernel(
        _deg_body,
        out_type=jax.ShapeDtypeStruct((NC, 2, NPAD), jnp.float32),
        mesh=_sc_mesh(),
        scratch_types=[
            pltpu.VMEM((CPW, CHUNK), jnp.int32),
            pltpu.VMEM((CPW, CHUNK), jnp.int32),
            pltpu.VMEM((CPW, CHUNK), jnp.float32),
            pltpu.VMEM((640,), jnp.float32),
            pltpu.VMEM_SHARED((NPAD,), jnp.float32),
            pltpu.VMEM_SHARED((NPAD,), jnp.float32),
        ],
    )
    return f(src2d, dst2d)


NPH = 2
IPP = CPW // NPH


def _agg_body(hn_hbm, src_hbm, dst_hbm, out_hbm, sidx, didx, rbuf,
              sem0, sem1, acc):
    c = lax.axis_index("c")
    s = lax.axis_index("s")
    wid = s * NC + c

    def _zfill(r, _):
        for i in range(D // 16):
            rbuf[0, r, pl.ds(i * 16, 16)] = jnp.zeros((16,), jnp.float32)
        return 0
    lax.fori_loop(0, CHUNK, _zfill, 0)
    for j in range(5):
        pltpu.sync_copy(rbuf.at[0], acc.at[pl.ds(s * 640 + j * CHUNK, CHUNK)])

    plsc.subcore_barrier()

    sems = (sem0, sem1)
    NB = 2

    for ph in range(NPH):
        base = wid * CPW + ph * IPP
        pltpu.sync_copy(src_hbm.at[pl.ds(base, IPP)], sidx)
        pltpu.sync_copy(dst_hbm.at[pl.ds(base, IPP)], didx)

        for b in range(NB):
            pltpu.async_copy(hn_hbm.at[sidx.at[b]], rbuf.at[b], sems[b])

        def _outer(jo, _):
            for b in range(NB):
                j = jo * NB + b
                pltpu.make_async_copy(hn_hbm.at[sidx.at[j]], rbuf.at[b],
                                      sems[b]).wait()
                pltpu.sync_copy(rbuf.at[b], acc.at[didx.at[j]], add=True)

                @pl.when(j + NB < IPP)
                def _():
                    pltpu.async_copy(hn_hbm.at[sidx.at[j + NB]], rbuf.at[b],
                                     sems[b])
            return 0
        lax.fori_loop(0, IPP // NB, _outer, 0)

    plsc.subcore_barrier()

    pltpu.sync_copy(acc.at[pl.ds(s * 640, 640)],
                    out_hbm.at[c, pl.ds(s * 640, 640)])


def _aggregate(hn, src2d, dst2d):
    f = pl.kernel(
        _agg_body,
        out_type=jax.ShapeDtypeStruct((NC, NPAD, D), jnp.float32),
        mesh=_sc_mesh(),
        scratch_types=[
            pltpu.VMEM((IPP, CHUNK), jnp.int32),
            pltpu.VMEM((IPP, CHUNK), jnp.int32),
            pltpu.VMEM((2, CHUNK, D), jnp.float32),
            pltpu.SemaphoreType.DMA,
            pltpu.SemaphoreType.DMA,
            pltpu.VMEM_SHARED((NPAD, D), jnp.float32),
        ],
    )
    return f(hn, src2d, dst2d)


def _norm_body(degp_ref, h_ref, hn_ref, cdst_ref):
    degp = degp_ref[...]
    deg_out = degp[0, 0] + degp[1, 0]
    deg_in = degp[0, 1] + degp[1, 1]
    csrc = lax.rsqrt(jnp.where(deg_out > 0, deg_out, 1.0))
    cdst_ref[...] = lax.rsqrt(jnp.where(deg_in > 0, deg_in, 1.0))
    hn_ref[pl.ds(0, N)] = h_ref[...] * csrc[:N]
    hn_ref[pl.ds(N, PAD_ROWS)] = jnp.zeros((PAD_ROWS, D), jnp.float32)


def _normalize(degp, h):
    return pl.pallas_call(
        _norm_body,
        out_shape=(jax.ShapeDtypeStruct((NPAD, D), jnp.float32),
                   jax.ShapeDtypeStruct((NPAD, 1), jnp.float32)),
    )(degp, h)


def _dense_body(aggp_ref, cdst_ref, h_ref, wf_ref, bf_ref, gf_ref, betf_ref,
                wp_ref, bp_ref, gp_ref, betp_ref, feat_ref, assign_ref):
    aggp = aggp_ref[...]
    scaled = (aggp[0] + aggp[1])[:N] * cdst_ref[...][:N]

    zf = jnp.dot(scaled, wf_ref[...], preferred_element_type=jnp.float32)
    zf = zf + bf_ref[...]
    m = jnp.mean(zf, axis=0, keepdims=True)
    xc = zf - m
    v = jnp.mean(xc * xc, axis=0, keepdims=True)
    featv = xc / jnp.sqrt(v + 1e-5) * gf_ref[...] + betf_ref[...]
    feat_ref[...] = jnp.maximum(featv, 0.0) + h_ref[...]

    zp = jnp.dot(scaled, wp_ref[...], preferred_element_type=jnp.float32)
    zp = zp + bp_ref[...]
    mp = jnp.mean(zp, axis=0, keepdims=True)
    xp = zp - mp
    vp = jnp.mean(xp * xp, axis=0, keepdims=True)
    ap = xp / jnp.sqrt(vp + 1e-5) * gp_ref[...] + betp_ref[...]
    ap = jnp.maximum(ap, 0.0)
    mx = jnp.max(ap, axis=1, keepdims=True)
    ex = jnp.exp(ap - mx)
    assign_ref[...] = ex / jnp.sum(ex, axis=1, keepdims=True)


def _dense(aggp, cdst, h, wf, bf, gf, betf, wp, bp, gp, betp):
    return pl.pallas_call(
        _dense_body,
        out_shape=(jax.ShapeDtypeStruct((N, D), jnp.float32),
                   jax.ShapeDtypeStruct((N, K), jnp.float32)),
    )(aggp, cdst, h, wf, bf, gf, betf, wp, bp, gp, betp)


def _pool_body(a_ref, f_ref, out_ref):
    out_ref[0] = lax.dot_general(a_ref[0], f_ref[0],
                                 (((0,), (0,)), ((), ())),
                                 preferred_element_type=jnp.float32)


def _pool(assign3, feat3):
    npg = N // BZ
    return pl.pallas_call(
        _pool_body,
        grid=(BZ,),
        in_specs=[pl.BlockSpec((1, npg, K), lambda b: (b, 0, 0)),
                  pl.BlockSpec((1, npg, D), lambda b: (b, 0, 0))],
        out_specs=pl.BlockSpec((1, K, D), lambda b: (b, 0, 0)),
        out_shape=jax.ShapeDtypeStruct((BZ, K, D), jnp.float32),
    )(assign3, feat3)


def _gram_body(p1_ref, p2_ref, out_ref):
    M = BZ * K
    inv = 1.0 / (M ** 0.5)
    out_ref[...] = lax.dot_general(p1_ref[...], p2_ref[...],
                                   (((1,), (1,)), ((), ())),
                                   preferred_element_type=jnp.float32) * inv


def _gram(pooled):
    M = BZ * K
    BLK = 128
    return pl.pallas_call(
        _gram_body,
        grid=(M // BLK,),
        in_specs=[pl.BlockSpec((BLK, D), lambda i: (i, 0)),
                  pl.BlockSpec((M, D), lambda i: (0, 0))],
        out_specs=pl.BlockSpec((BLK, M), lambda i: (i, 0)),
        out_shape=jax.ShapeDtypeStruct((M, M), jnp.float32),
    )(pooled, pooled)


def kernel(h, edge_index, e, W_feat, b_feat, gamma_feat, beta_feat,
           W_pool, b_pool, gamma_pool, beta_pool):
    src = edge_index[0]
    dst = edge_index[1]
    npad_e = EPAD - E
    pad_idx = N + (jnp.arange(npad_e, dtype=jnp.int32) % PAD_ROWS)
    src2d = jnp.concatenate([src, pad_idx]).reshape(NCHUNKS, CHUNK)
    dst2d = jnp.concatenate([dst, pad_idx]).reshape(NCHUNKS, CHUNK)

    degp = _degrees(src2d, dst2d)
    hn, cdst = _normalize(degp.reshape(NC, 2, NPAD, 1), h)
    aggp = _aggregate(hn, src2d, dst2d)
    feat, assign = _dense(
        aggp, cdst, h,
        W_feat, b_feat.reshape(1, D), gamma_feat.reshape(1, D),
        beta_feat.reshape(1, D),
        W_pool, b_pool.reshape(1, K), gamma_pool.reshape(1, K),
        beta_pool.reshape(1, K))
    pooled = _pool(assign.reshape(BZ, N // BZ, K),
                   feat.reshape(BZ, N // BZ, D))
    return _gram(pooled.reshape(BZ * K, D))


# speedup vs baseline: 11.2769x; 1.3440x over previous
import functools

import jax
import jax.numpy as jnp
from jax import lax
from jax.experimental import pallas as pl
from jax.experimental.pallas import tpu as pltpu
from jax.experimental.pallas import tpu_sc as plsc

N = 10000
E = 320000
D = 128
K = 32
BZ = 100

NC = 2
NS = 16
NW = NC * NS
NPAD = 10240
CHUNK = 128
NCHUNKS = 2560
EPAD = NCHUNKS * CHUNK
CPW = NCHUNKS // NW
PAD_ROWS = NPAD - N


def _sc_mesh():
    return plsc.VectorSubcoreMesh(core_axis_name="c", subcore_axis_name="s")


def _deg_body(src_hbm, dst_hbm, out_hbm, sidx, didx, ones_v, zbuf,
              hist_out, hist_in):
    c = lax.axis_index("c")
    s = lax.axis_index("s")
    wid = s * NC + c

    def _zfill(i, _):
        i16 = pl.multiple_of(i * 16, 16)
        zbuf[pl.ds(i16, 16)] = jnp.zeros((16,), jnp.float32)
        return 0
    lax.fori_loop(0, 640 // 16, _zfill, 0)

    def _ofill(i, _):
        i16 = pl.multiple_of(i * 16, 16)
        ones_v[pl.ds(i16, 16)] = jnp.ones((16,), jnp.float32)
        return 0
    lax.fori_loop(0, (CPW * CHUNK) // 16, _ofill, 0)

    pltpu.sync_copy(zbuf, hist_out.at[pl.ds(s * 640, 640)])
    pltpu.sync_copy(zbuf, hist_in.at[pl.ds(s * 640, 640)])

    pltpu.sync_copy(src_hbm.at[pl.ds(wid * (CPW * CHUNK), CPW * CHUNK)], sidx)
    pltpu.sync_copy(dst_hbm.at[pl.ds(wid * (CPW * CHUNK), CPW * CHUNK)], didx)

    plsc.subcore_barrier()

    pltpu.sync_copy(ones_v, hist_out.at[sidx], add=True)
    pltpu.sync_copy(ones_v, hist_in.at[didx], add=True)

    plsc.subcore_barrier()

    pltpu.sync_copy(hist_out.at[pl.ds(s * 640, 640)],
                    out_hbm.at[c, 0, pl.ds(s * 640, 640)])
    pltpu.sync_copy(hist_in.at[pl.ds(s * 640, 640)],
                    out_hbm.at[c, 1, pl.ds(s * 640, 640)])


def _degrees(src2d, dst2d):
    f = pl.kernel(
        _deg_body,
        out_type=jax.ShapeDtypeStruct((NC, 2, NPAD), jnp.float32),
        mesh=_sc_mesh(),
        scratch_types=[
            pltpu.VMEM((CPW * CHUNK,), jnp.int32),
            pltpu.VMEM((CPW * CHUNK,), jnp.int32),
            pltpu.VMEM((CPW * CHUNK,), jnp.float32),
            pltpu.VMEM((640,), jnp.float32),
            pltpu.VMEM_SHARED((NPAD,), jnp.float32),
            pltpu.VMEM_SHARED((NPAD,), jnp.float32),
        ],
    )
    return f(src2d.reshape(EPAD), dst2d.reshape(EPAD))


NPH = 2
IPP = CPW // NPH


def _agg_body(hn_hbm, src_hbm, dst_hbm, out_hbm, sidx, didx, rbuf,
              sem0, sem1, acc):
    c = lax.axis_index("c")
    s = lax.axis_index("s")
    wid = s * NC + c

    def _zfill(r, _):
        for i in range(D // 16):
            rbuf[0, r, pl.ds(i * 16, 16)] = jnp.zeros((16,), jnp.float32)
        return 0
    lax.fori_loop(0, CHUNK, _zfill, 0)
    for j in range(5):
        pltpu.sync_copy(rbuf.at[0], acc.at[pl.ds(s * 640 + j * CHUNK, CHUNK)])

    plsc.subcore_barrier()

    sems = (sem0, sem1)
    NB = 2

    for ph in range(NPH):
        base = wid * CPW + ph * IPP
        pltpu.sync_copy(src_hbm.at[pl.ds(base, IPP)], sidx)
        pltpu.sync_copy(dst_hbm.at[pl.ds(base, IPP)], didx)

        for b in range(NB):
            pltpu.async_copy(hn_hbm.at[sidx.at[b]], rbuf.at[b], sems[b])

        def _outer(jo, _):
            for b in range(NB):
                j = jo * NB + b
                pltpu.make_async_copy(hn_hbm.at[sidx.at[j]], rbuf.at[b],
                                      sems[b]).wait()
                pltpu.sync_copy(rbuf.at[b], acc.at[didx.at[j]], add=True)

                @pl.when(j + NB < IPP)
                def _():
                    pltpu.async_copy(hn_hbm.at[sidx.at[j + NB]], rbuf.at[b],
                                     sems[b])
            return 0
        lax.fori_loop(0, IPP // NB, _outer, 0)

    plsc.subcore_barrier()

    pltpu.sync_copy(acc.at[pl.ds(s * 640, 640)],
                    out_hbm.at[c, pl.ds(s * 640, 640)])


def _aggregate(hn, src2d, dst2d):
    f = pl.kernel(
        _agg_body,
        out_type=jax.ShapeDtypeStruct((NC, NPAD, D), jnp.float32),
        mesh=_sc_mesh(),
        scratch_types=[
            pltpu.VMEM((IPP, CHUNK), jnp.int32),
            pltpu.VMEM((IPP, CHUNK), jnp.int32),
            pltpu.VMEM((2, CHUNK, D), jnp.float32),
            pltpu.SemaphoreType.DMA,
            pltpu.SemaphoreType.DMA,
            pltpu.VMEM_SHARED((NPAD, D), jnp.float32),
        ],
    )
    return f(hn, src2d, dst2d)


def _norm_body(degp_ref, h_ref, hn_ref, cdst_ref):
    degp = degp_ref[...]
    deg_out = degp[0, 0] + degp[1, 0]
    deg_in = degp[0, 1] + degp[1, 1]
    csrc = lax.rsqrt(jnp.where(deg_out > 0, deg_out, 1.0))
    cdst_ref[...] = lax.rsqrt(jnp.where(deg_in > 0, deg_in, 1.0))
    hn_ref[pl.ds(0, N)] = h_ref[...] * csrc[:N]
    hn_ref[pl.ds(N, PAD_ROWS)] = jnp.zeros((PAD_ROWS, D), jnp.float32)


def _normalize(degp, h):
    return pl.pallas_call(
        _norm_body,
        out_shape=(jax.ShapeDtypeStruct((NPAD, D), jnp.float32),
                   jax.ShapeDtypeStruct((NPAD, 1), jnp.float32)),
    )(degp, h)


def _dense_body(aggp_ref, cdst_ref, h_ref, wf_ref, bf_ref, gf_ref, betf_ref,
                wp_ref, bp_ref, gp_ref, betp_ref, pooled_ref):
    aggp = aggp_ref[...]
    scaled = (aggp[0] + aggp[1])[:N] * cdst_ref[...][:N]

    zf = jnp.dot(scaled, wf_ref[...], preferred_element_type=jnp.float32)
    zf = zf + bf_ref[...]
    m = jnp.mean(zf, axis=0, keepdims=True)
    xc = zf - m
    v = jnp.mean(xc * xc, axis=0, keepdims=True)
    featv = xc / jnp.sqrt(v + 1e-5) * gf_ref[...] + betf_ref[...]
    feat = jnp.maximum(featv, 0.0) + h_ref[...]

    zp = jnp.dot(scaled, wp_ref[...], preferred_element_type=jnp.float32)
    zp = zp + bp_ref[...]
    mp = jnp.mean(zp, axis=0, keepdims=True)
    xp = zp - mp
    vp = jnp.mean(xp * xp, axis=0, keepdims=True)
    ap = xp / jnp.sqrt(vp + 1e-5) * gp_ref[...] + betp_ref[...]
    ap = jnp.maximum(ap, 0.0)
    mx = jnp.max(ap, axis=1, keepdims=True)
    ex = jnp.exp(ap - mx)
    assign = ex / jnp.sum(ex, axis=1, keepdims=True)

    feat3 = feat.reshape(BZ, N // BZ, D)
    assign3 = assign.reshape(BZ, N // BZ, K)
    pooled_ref[...] = lax.dot_general(
        assign3, feat3, (((1,), (1,)), ((0,), (0,))),
        preferred_element_type=jnp.float32)


def _dense(aggp, cdst, h, wf, bf, gf, betf, wp, bp, gp, betp):
    return pl.pallas_call(
        _dense_body,
        out_shape=jax.ShapeDtypeStruct((BZ, K, D), jnp.float32),
    )(aggp, cdst, h, wf, bf, gf, betf, wp, bp, gp, betp)


def _pool_body(a_ref, f_ref, out_ref):
    out_ref[0] = lax.dot_general(a_ref[0], f_ref[0],
                                 (((0,), (0,)), ((), ())),
                                 preferred_element_type=jnp.float32)


def _pool(assign3, feat3):
    npg = N // BZ
    return pl.pallas_call(
        _pool_body,
        grid=(BZ,),
        in_specs=[pl.BlockSpec((1, npg, K), lambda b: (b, 0, 0)),
                  pl.BlockSpec((1, npg, D), lambda b: (b, 0, 0))],
        out_specs=pl.BlockSpec((1, K, D), lambda b: (b, 0, 0)),
        out_shape=jax.ShapeDtypeStruct((BZ, K, D), jnp.float32),
    )(assign3, feat3)


def _gram_body(p1_ref, p2_ref, out_ref):
    M = BZ * K
    inv = 1.0 / (M ** 0.5)
    out_ref[...] = lax.dot_general(p1_ref[...], p2_ref[...],
                                   (((1,), (1,)), ((), ())),
                                   preferred_element_type=jnp.float32) * inv


def _gram(pooled):
    M = BZ * K
    BLK = 128
    return pl.pallas_call(
        _gram_body,
        grid=(M // BLK,),
        in_specs=[pl.BlockSpec((BLK, D), lambda i: (i, 0)),
                  pl.BlockSpec((M, D), lambda i: (0, 0))],
        out_specs=pl.BlockSpec((BLK, M), lambda i: (i, 0)),
        out_shape=jax.ShapeDtypeStruct((M, M), jnp.float32),
    )(pooled, pooled)


def kernel(h, edge_index, e, W_feat, b_feat, gamma_feat, beta_feat,
           W_pool, b_pool, gamma_pool, beta_pool):
    src = edge_index[0]
    dst = edge_index[1]
    npad_e = EPAD - E
    pad_idx = N + (jnp.arange(npad_e, dtype=jnp.int32) % PAD_ROWS)
    src2d = jnp.concatenate([src, pad_idx]).reshape(NCHUNKS, CHUNK)
    dst2d = jnp.concatenate([dst, pad_idx]).reshape(NCHUNKS, CHUNK)

    degp = _degrees(src2d, dst2d)
    hn, cdst = _normalize(degp.reshape(NC, 2, NPAD, 1), h)
    aggp = _aggregate(hn, src2d, dst2d)
    pooled = _dense(
        aggp, cdst, h,
        W_feat, b_feat.reshape(1, D), gamma_feat.reshape(1, D),
        beta_feat.reshape(1, D),
        W_pool, b_pool.reshape(1, K), gamma_pool.reshape(1, K),
        beta_pool.reshape(1, K))
    return _gram(pooled.reshape(BZ * K, D))
